# baseline (speedup 1.0000x reference)
import functools

import jax
import jax.numpy as jnp
from jax import lax
from jax.experimental import pallas as pl
from jax.experimental.pallas import tpu as pltpu
from jax.experimental.pallas import tpu_sc as plsc

VOCAB = 100000
D_MODEL = 64
BATCH = 1024

_VBLK = 1024


def _sc_gather(emb_table, input_ids):
    info = plsc.get_sparse_core_info()
    nw = info.num_cores * info.num_subcores
    b_per_w = BATCH // nw
    mesh = plsc.VectorSubcoreMesh(core_axis_name="c", subcore_axis_name="s")

    @functools.partial(
        pl.kernel,
        mesh=mesh,
        out_type=jax.ShapeDtypeStruct((BATCH, D_MODEL), jnp.float32),
        compiler_params=pltpu.CompilerParams(use_tc_tiling_on_sc=False),
        scratch_types=[
            pltpu.VMEM((b_per_w,), jnp.int32),
            pltpu.VMEM((b_per_w, D_MODEL), jnp.float32),
            pltpu.SemaphoreType.DMA,
        ],
    )
    def gather_kernel(table_hbm, idx_hbm, out_hbm, idx_v, rows_v, sem):
        wid = lax.axis_index("s") * info.num_cores + lax.axis_index("c")
        base = wid * b_per_w
        pltpu.sync_copy(idx_hbm.at[pl.ds(base, b_per_w)], idx_v)
        pltpu.async_copy(table_hbm.at[idx_v], rows_v, sem).wait()
        pltpu.sync_copy(rows_v, out_hbm.at[pl.ds(base, b_per_w)])

    return gather_kernel(emb_table, input_ids)


def _tc_project(x, proj_w):
    def mm(x_ref, w_ref, o_ref):
        o_ref[...] = lax.dot_general(
            x_ref[...],
            w_ref[...],
            (((1,), (1,)), ((), ())),
            preferred_element_type=jnp.float32,
        )

    return pl.pallas_call(
        mm,
        grid=(pl.cdiv(VOCAB, _VBLK),),
        in_specs=[
            pl.BlockSpec((BATCH, D_MODEL), lambda i: (0, 0)),
            pl.BlockSpec((_VBLK, D_MODEL), lambda i: (i, 0)),
        ],
        out_specs=pl.BlockSpec((BATCH, _VBLK), lambda i: (0, i)),
        out_shape=jax.ShapeDtypeStruct((BATCH, VOCAB), jnp.float32),
    )(x, proj_w)


def kernel(input_ids, emb_table, proj_w):
    x = _sc_gather(emb_table, input_ids.astype(jnp.int32))
    return _tc_project(x, proj_w)


# speedup vs baseline: 1.0304x; 1.0304x over previous
import functools

import jax
import jax.numpy as jnp
from jax import lax
from jax.experimental import pallas as pl
from jax.experimental.pallas import tpu as pltpu
from jax.experimental.pallas import tpu_sc as plsc

VOCAB = 100000
D_MODEL = 64
BATCH = 1024

_VBLK = 1024


def _sc_gather(emb_table, input_ids):
    info = plsc.get_sparse_core_info()
    nw = info.num_cores * info.num_subcores
    b_per_w = BATCH // nw
    mesh = plsc.VectorSubcoreMesh(core_axis_name="c", subcore_axis_name="s")

    @functools.partial(
        pl.kernel,
        mesh=mesh,
        out_type=jax.ShapeDtypeStruct((BATCH, D_MODEL), jnp.float32),
        compiler_params=pltpu.CompilerParams(use_tc_tiling_on_sc=False),
        scratch_types=[
            pltpu.VMEM((b_per_w,), jnp.int32),
            pltpu.VMEM((b_per_w, D_MODEL), jnp.float32),
            pltpu.SemaphoreType.DMA,
        ],
    )
    def gather_kernel(table_hbm, idx_hbm, out_hbm, idx_v, rows_v, sem):
        wid = lax.axis_index("s") * info.num_cores + lax.axis_index("c")
        base = wid * b_per_w
        pltpu.sync_copy(idx_hbm.at[pl.ds(base, b_per_w)], idx_v)
        pltpu.async_copy(table_hbm.at[idx_v], rows_v, sem).wait()
        pltpu.sync_copy(rows_v, out_hbm.at[pl.ds(base, b_per_w)])

    return gather_kernel(emb_table, input_ids)


_NBUF = 4
_NSTEPS = (VOCAB + _VBLK - 1) // _VBLK
_TAIL = VOCAB - (_NSTEPS - 1) * _VBLK


def _tc_project(x, proj_w):
    def mm(x_ref, w_ref, o_hbm, acc, tail_buf, sems, tail_sem):
        i = pl.program_id(0)

        def dot(xv, wv):
            return lax.dot_general(
                xv, wv, (((1,), (1,)), ((), ())),
                preferred_element_type=jnp.float32,
            )

        for b in range(_NBUF):
            @pl.when(jnp.logical_and(i % _NBUF == b, i >= _NBUF))
            def _():
                pltpu.make_async_copy(
                    acc.at[b],
                    o_hbm.at[:, pl.ds((i - _NBUF) * _VBLK, _VBLK)],
                    sems.at[b],
                ).wait()

            @pl.when(jnp.logical_and(i % _NBUF == b, i < _NSTEPS - 1))
            def _():
                acc[b, :, :] = dot(x_ref[...], w_ref[...])

            @pl.when(jnp.logical_and(i % _NBUF == b, i < _NSTEPS - 1))
            def _():
                pltpu.make_async_copy(
                    acc.at[b],
                    o_hbm.at[:, pl.ds(i * _VBLK, _VBLK)],
                    sems.at[b],
                ).start()

        @pl.when(i == _NSTEPS - 1)
        def _():
            tail_buf[...] = dot(x_ref[...], w_ref[...])[:, :_TAIL]
            tail = pltpu.make_async_copy(
                tail_buf,
                o_hbm.at[:, pl.ds((_NSTEPS - 1) * _VBLK, _TAIL)],
                tail_sem,
            )
            tail.start()
            for s in range(_NSTEPS - _NBUF, _NSTEPS - 1):
                b = s % _NBUF
                pltpu.make_async_copy(
                    acc.at[b],
                    o_hbm.at[:, pl.ds(s * _VBLK, _VBLK)],
                    sems.at[b],
                ).wait()
            tail.wait()

    return pl.pallas_call(
        mm,
        grid=(_NSTEPS,),
        in_specs=[
            pl.BlockSpec((BATCH, D_MODEL), lambda i: (0, 0)),
            pl.BlockSpec((_VBLK, D_MODEL), lambda i: (i, 0)),
        ],
        out_specs=pl.BlockSpec(memory_space=pl.ANY),
        out_shape=jax.ShapeDtypeStruct((BATCH, VOCAB), jnp.float32),
        scratch_shapes=[
            pltpu.VMEM((_NBUF, BATCH, _VBLK), jnp.float32),
            pltpu.VMEM((BATCH, _TAIL), jnp.float32),
            pltpu.SemaphoreType.DMA((_NBUF,)),
            pltpu.SemaphoreType.DMA,
        ],
    )(x, proj_w)


def kernel(input_ids, emb_table, proj_w):
    x = _sc_gather(emb_table, input_ids.astype(jnp.int32))
    return _tc_project(x, proj_w)


# speedup vs baseline: 1.0371x; 1.0065x over previous
import functools

import jax
import jax.numpy as jnp
from jax import lax
from jax.experimental import pallas as pl
from jax.experimental.pallas import tpu as pltpu
from jax.experimental.pallas import tpu_sc as plsc

VOCAB = 100000
D_MODEL = 64
BATCH = 1024

_VBLK = 2048


def _sc_gather(emb_table, input_ids):
    info = plsc.get_sparse_core_info()
    nw = info.num_cores * info.num_subcores
    b_per_w = BATCH // nw
    mesh = plsc.VectorSubcoreMesh(core_axis_name="c", subcore_axis_name="s")

    @functools.partial(
        pl.kernel,
        mesh=mesh,
        out_type=jax.ShapeDtypeStruct((BATCH, D_MODEL), jnp.float32),
        compiler_params=pltpu.CompilerParams(use_tc_tiling_on_sc=False),
        scratch_types=[
            pltpu.VMEM((b_per_w,), jnp.int32),
            pltpu.VMEM((b_per_w, D_MODEL), jnp.float32),
            pltpu.SemaphoreType.DMA,
        ],
    )
    def gather_kernel(table_hbm, idx_hbm, out_hbm, idx_v, rows_v, sem):
        wid = lax.axis_index("s") * info.num_cores + lax.axis_index("c")
        base = wid * b_per_w
        pltpu.sync_copy(idx_hbm.at[pl.ds(base, b_per_w)], idx_v)
        pltpu.async_copy(table_hbm.at[idx_v], rows_v, sem).wait()
        pltpu.sync_copy(rows_v, out_hbm.at[pl.ds(base, b_per_w)])

    return gather_kernel(emb_table, input_ids)


_NBUF = 2
_NSTEPS = (VOCAB + _VBLK - 1) // _VBLK
_TAIL = VOCAB - (_NSTEPS - 1) * _VBLK


def _tc_project(x, proj_w):
    def mm(x_ref, w_ref, o_hbm, acc, tail_buf, sems, tail_sem):
        i = pl.program_id(0)

        def dot(xv, wv):
            return lax.dot_general(
                xv, wv, (((1,), (1,)), ((), ())),
                preferred_element_type=jnp.float32,
            )

        for b in range(_NBUF):
            @pl.when(jnp.logical_and(i % _NBUF == b, i >= _NBUF))
            def _():
                pltpu.make_async_copy(
                    acc.at[b],
                    o_hbm.at[:, pl.ds((i - _NBUF) * _VBLK, _VBLK)],
                    sems.at[b],
                ).wait()

            @pl.when(jnp.logical_and(i % _NBUF == b, i < _NSTEPS - 1))
            def _():
                acc[b, :, :] = dot(x_ref[...], w_ref[...])

            @pl.when(jnp.logical_and(i % _NBUF == b, i < _NSTEPS - 1))
            def _():
                pltpu.make_async_copy(
                    acc.at[b],
                    o_hbm.at[:, pl.ds(i * _VBLK, _VBLK)],
                    sems.at[b],
                ).start()

        @pl.when(i == _NSTEPS - 1)
        def _():
            tail_buf[...] = dot(x_ref[...], w_ref[...])[:, :_TAIL]
            tail = pltpu.make_async_copy(
                tail_buf,
                o_hbm.at[:, pl.ds((_NSTEPS - 1) * _VBLK, _TAIL)],
                tail_sem,
            )
            tail.start()
            for s in range(_NSTEPS - _NBUF, _NSTEPS - 1):
                b = s % _NBUF
                pltpu.make_async_copy(
                    acc.at[b],
                    o_hbm.at[:, pl.ds(s * _VBLK, _VBLK)],
                    sems.at[b],
                ).wait()
            tail.wait()

    return pl.pallas_call(
        mm,
        grid=(_NSTEPS,),
        in_specs=[
            pl.BlockSpec((BATCH, D_MODEL), lambda i: (0, 0)),
            pl.BlockSpec((_VBLK, D_MODEL), lambda i: (i, 0)),
        ],
        out_specs=pl.BlockSpec(memory_space=pl.ANY),
        out_shape=jax.ShapeDtypeStruct((BATCH, VOCAB), jnp.float32),
        scratch_shapes=[
            pltpu.VMEM((_NBUF, BATCH, _VBLK), jnp.float32),
            pltpu.VMEM((BATCH, _TAIL), jnp.float32),
            pltpu.SemaphoreType.DMA((_NBUF,)),
            pltpu.SemaphoreType.DMA,
        ],
    )(x, proj_w)


def kernel(input_ids, emb_table, proj_w):
    x = _sc_gather(emb_table, input_ids.astype(jnp.int32))
    return _tc_project(x, proj_w)


# speedup vs baseline: 2.4376x; 2.3504x over previous
import functools

import jax
import jax.numpy as jnp
from jax import lax
from jax.experimental import pallas as pl
from jax.experimental.pallas import tpu as pltpu
from jax.experimental.pallas import tpu_sc as plsc

VOCAB = 100000
D_MODEL = 64
BATCH = 1024

_VBLK = 2048


def _sc_gather(emb_table, input_ids):
    info = plsc.get_sparse_core_info()
    nw = info.num_cores * info.num_subcores
    b_per_w = BATCH // nw
    mesh = plsc.VectorSubcoreMesh(core_axis_name="c", subcore_axis_name="s")

    @functools.partial(
        pl.kernel,
        mesh=mesh,
        out_type=jax.ShapeDtypeStruct((BATCH, D_MODEL), jnp.float32),
        compiler_params=pltpu.CompilerParams(use_tc_tiling_on_sc=False),
        scratch_types=[
            pltpu.VMEM((b_per_w,), jnp.int32),
            pltpu.VMEM((b_per_w, D_MODEL), jnp.float32),
            pltpu.SemaphoreType.DMA,
        ],
    )
    def gather_kernel(table_hbm, idx_hbm, out_hbm, idx_v, rows_v, sem):
        wid = lax.axis_index("s") * info.num_cores + lax.axis_index("c")
        base = wid * b_per_w
        pltpu.sync_copy(idx_hbm.at[pl.ds(base, b_per_w)], idx_v)
        pltpu.async_copy(table_hbm.at[idx_v], rows_v, sem).wait()
        pltpu.sync_copy(rows_v, out_hbm.at[pl.ds(base, b_per_w)])

    return gather_kernel(emb_table, input_ids)


_NSTEPS = (VOCAB + _VBLK - 1) // _VBLK


def _tc_project_t(x, proj_w):
    def mm(x_ref, w_ref, o_ref):
        o_ref[...] = lax.dot_general(
            w_ref[...],
            x_ref[...],
            (((1,), (1,)), ((), ())),
            preferred_element_type=jnp.float32,
        )

    return pl.pallas_call(
        mm,
        grid=(_NSTEPS,),
        in_specs=[
            pl.BlockSpec((BATCH, D_MODEL), lambda i: (0, 0)),
            pl.BlockSpec((_VBLK, D_MODEL), lambda i: (i, 0)),
        ],
        out_specs=pl.BlockSpec((_VBLK, BATCH), lambda i: (i, 0)),
        out_shape=jax.ShapeDtypeStruct((VOCAB, BATCH), jnp.float32),
    )(x, proj_w)


def kernel(input_ids, emb_table, proj_w):
    x = _sc_gather(emb_table, input_ids.astype(jnp.int32))
    return _tc_project_t(x, proj_w).T


# speedup vs baseline: 2.9310x; 1.2024x over previous
import functools

import jax
import jax.numpy as jnp
from jax import lax
from jax.experimental import pallas as pl
from jax.experimental.pallas import tpu as pltpu
from jax.experimental.pallas import tpu_sc as plsc

VOCAB = 100000
D_MODEL = 64
BATCH = 1024

_VBLK = 2048


def _sc_gather(emb_table, input_ids):
    info = plsc.get_sparse_core_info()
    nw = info.num_cores * info.num_subcores
    b_per_w = BATCH // nw
    mesh = plsc.VectorSubcoreMesh(core_axis_name="c", subcore_axis_name="s")

    @functools.partial(
        pl.kernel,
        mesh=mesh,
        out_type=jax.ShapeDtypeStruct((BATCH, D_MODEL), jnp.float32),
        compiler_params=pltpu.CompilerParams(use_tc_tiling_on_sc=False),
        scratch_types=[
            pltpu.VMEM((b_per_w,), jnp.int32),
            pltpu.VMEM((b_per_w, D_MODEL), jnp.float32),
            pltpu.SemaphoreType.DMA,
        ],
    )
    def gather_kernel(table_hbm, idx_hbm, out_hbm, idx_v, rows_v, sem):
        wid = lax.axis_index("s") * info.num_cores + lax.axis_index("c")
        base = wid * b_per_w
        pltpu.sync_copy(idx_hbm.at[pl.ds(base, b_per_w)], idx_v)
        pltpu.async_copy(table_hbm.at[idx_v], rows_v, sem).wait()
        pltpu.sync_copy(rows_v, out_hbm.at[pl.ds(base, b_per_w)])

    return gather_kernel(emb_table, input_ids)


_NSTEPS = (VOCAB + _VBLK - 1) // _VBLK


def _tc_project_t(x, proj_w):
    def mm(x_ref, wt_ref, o_ref):
        o_ref[...] = lax.dot_general(
            wt_ref[...],
            x_ref[...],
            (((0,), (1,)), ((), ())),
            preferred_element_type=jnp.float32,
        )

    return pl.pallas_call(
        mm,
        grid=(_NSTEPS,),
        in_specs=[
            pl.BlockSpec((BATCH, D_MODEL), lambda i: (0, 0)),
            pl.BlockSpec((D_MODEL, _VBLK), lambda i: (0, i)),
        ],
        out_specs=pl.BlockSpec((_VBLK, BATCH), lambda i: (i, 0)),
        out_shape=jax.ShapeDtypeStruct((VOCAB, BATCH), jnp.float32),
    )(x, proj_w.T)


def kernel(input_ids, emb_table, proj_w):
    x = _sc_gather(emb_table, input_ids.astype(jnp.int32))
    return _tc_project_t(x, proj_w).T


# speedup vs baseline: 2.9527x; 1.0074x over previous
import functools

import jax
import jax.numpy as jnp
from jax import lax
from jax.experimental import pallas as pl
from jax.experimental.pallas import tpu as pltpu
from jax.experimental.pallas import tpu_sc as plsc

VOCAB = 100000
D_MODEL = 64
BATCH = 1024

_VBLK = 4096


def _sc_gather(emb_table, input_ids):
    info = plsc.get_sparse_core_info()
    nw = info.num_cores * info.num_subcores
    b_per_w = BATCH // nw
    mesh = plsc.VectorSubcoreMesh(core_axis_name="c", subcore_axis_name="s")

    @functools.partial(
        pl.kernel,
        mesh=mesh,
        out_type=jax.ShapeDtypeStruct((BATCH, D_MODEL), jnp.float32),
        compiler_params=pltpu.CompilerParams(use_tc_tiling_on_sc=False),
        scratch_types=[
            pltpu.VMEM((b_per_w,), jnp.int32),
            pltpu.VMEM((b_per_w, D_MODEL), jnp.float32),
            pltpu.SemaphoreType.DMA,
        ],
    )
    def gather_kernel(table_hbm, idx_hbm, out_hbm, idx_v, rows_v, sem):
        wid = lax.axis_index("s") * info.num_cores + lax.axis_index("c")
        base = wid * b_per_w
        pltpu.sync_copy(idx_hbm.at[pl.ds(base, b_per_w)], idx_v)
        pltpu.async_copy(table_hbm.at[idx_v], rows_v, sem).wait()
        pltpu.sync_copy(rows_v, out_hbm.at[pl.ds(base, b_per_w)])

    return gather_kernel(emb_table, input_ids)


_NSTEPS = (VOCAB + _VBLK - 1) // _VBLK


def _tc_project_t(x, proj_w):
    def mm(x_ref, wt_ref, o_ref):
        o_ref[...] = lax.dot_general(
            wt_ref[...],
            x_ref[...],
            (((0,), (1,)), ((), ())),
            preferred_element_type=jnp.float32,
        )

    return pl.pallas_call(
        mm,
        grid=(_NSTEPS,),
        in_specs=[
            pl.BlockSpec((BATCH, D_MODEL), lambda i: (0, 0)),
            pl.BlockSpec((D_MODEL, _VBLK), lambda i: (0, i)),
        ],
        out_specs=pl.BlockSpec((_VBLK, BATCH), lambda i: (i, 0)),
        out_shape=jax.ShapeDtypeStruct((VOCAB, BATCH), jnp.float32),
    )(x, proj_w.T)


def kernel(input_ids, emb_table, proj_w):
    x = _sc_gather(emb_table, input_ids.astype(jnp.int32))
    return _tc_project_t(x, proj_w).T
